# baseline (speedup 1.0000x reference)
import functools

import jax
import jax.numpy as jnp
from jax.experimental import pallas as pl
from jax.experimental.pallas import tpu as pltpu

E = 8
DH = 128
DM = 2048
TILE = 512
KAUG = E * DH + 128


def _body(x_ref, sel_ref, p_ref, w_ref, b_ref, o_ref, ws_ref):
    @pl.when(pl.program_id(0) == 0)
    def _init():
        def cast_chunk(k, _):
            ws_ref[pl.ds(k * DH, DH), :] = (
                w_ref[pl.ds(k * DH, DH), :].astype(jnp.bfloat16))
            return 0
        jax.lax.fori_loop(0, E, cast_chunk, 0)
        ws_ref[pl.ds(E * DH, DH), :] = jnp.concatenate(
            [b_ref[...], jnp.zeros((DH - E, DM), jnp.float32)], axis=0
        ).astype(jnp.bfloat16)

    HT = TILE // 2
    for h in range(2):
        rows = pl.ds(h * HT, HT)
        x = x_ref[rows, :]
        sel = sel_ref[rows, :]
        p = p_ref[rows, :]
        p0 = p[:, 0:1]
        p1 = p[:, 1:2]
        zero = jnp.zeros((), jnp.bfloat16)
        x0 = (x[:, :DH] * p0).astype(jnp.bfloat16)
        x1 = (x[:, DH:] * p1).astype(jnp.bfloat16)
        s0 = sel[:, 0:1].astype(jnp.bfloat16)
        s1 = sel[:, 1:2].astype(jnp.bfloat16)
        zs = []
        for e in range(E):
            eb = jnp.full((), e, jnp.bfloat16)
            zs.append(jnp.where(s0 == eb, x0, zero)
                      + jnp.where(s1 == eb, x1, zero))
        lane = jax.lax.broadcasted_iota(jnp.int32, (HT, DH), 1)
        q = jnp.where(lane == sel[:, 0:1], p0, 0.0) + jnp.where(
            lane == sel[:, 1:2], p1, 0.0)
        zs.append(q.astype(jnp.bfloat16))
        z = jnp.concatenate(zs, axis=1)
        o_ref[rows, :] = jnp.dot(z, ws_ref[...],
                                 preferred_element_type=jnp.float32)


@functools.partial(jax.jit, static_argnames=("interpret",))
def kernel(tensor, head_selection, head_probabilities, W, b, interpret=False):
    B, S, H, dh = tensor.shape
    N = B * S
    x = tensor.reshape(N, H * dh)
    sel = head_selection.reshape(N, H)
    p = head_probabilities.reshape(N, H)
    w2 = W.reshape(E * DH, DM)

    out = pl.pallas_call(
        _body,
        grid=(N // TILE,),
        in_specs=[
            pl.BlockSpec((TILE, H * dh), lambda i: (i, 0)),
            pl.BlockSpec((TILE, H), lambda i: (i, 0)),
            pl.BlockSpec((TILE, H), lambda i: (i, 0)),
            pl.BlockSpec((E * DH, DM), lambda i: (0, 0)),
            pl.BlockSpec((E, DM), lambda i: (0, 0)),
        ],
        out_specs=pl.BlockSpec((TILE, DM), lambda i: (i, 0)),
        out_shape=jax.ShapeDtypeStruct((N, DM), jnp.float32),
        scratch_shapes=[pltpu.VMEM((KAUG, DM), jnp.bfloat16)],
        interpret=interpret,
    )(x, sel, p, w2, b)
    return out.reshape(B, S, DM)


# speedup vs baseline: 1.0427x; 1.0427x over previous
import functools

import jax
import jax.numpy as jnp
from jax.experimental import pallas as pl
from jax.experimental.pallas import tpu as pltpu

E = 8
DH = 128
DM = 2048
TILE = 1024
KAUG = E * DH + 128


def _body(x_ref, sel_ref, p_ref, w_ref, b_ref, o_ref, ws_ref):
    @pl.when(pl.program_id(0) == 0)
    def _init():
        def cast_chunk(k, _):
            ws_ref[pl.ds(k * DH, DH), :] = (
                w_ref[pl.ds(k * DH, DH), :].astype(jnp.bfloat16))
            return 0
        jax.lax.fori_loop(0, E, cast_chunk, 0)
        ws_ref[pl.ds(E * DH, DH), :] = jnp.concatenate(
            [b_ref[...], jnp.zeros((DH - E, DM), jnp.float32)], axis=0
        ).astype(jnp.bfloat16)

    HT = TILE // 2
    for h in range(2):
        rows = pl.ds(h * HT, HT)
        x = x_ref[rows, :]
        sel = sel_ref[rows, :]
        p = p_ref[rows, :]
        p0 = p[:, 0:1]
        p1 = p[:, 1:2]
        zero = jnp.zeros((), jnp.bfloat16)
        x0 = (x[:, :DH] * p0).astype(jnp.bfloat16)
        x1 = (x[:, DH:] * p1).astype(jnp.bfloat16)
        s0 = sel[:, 0:1].astype(jnp.bfloat16)
        s1 = sel[:, 1:2].astype(jnp.bfloat16)
        zs = []
        for e in range(E):
            eb = jnp.full((), e, jnp.bfloat16)
            zs.append(jnp.where(s0 == eb, x0, zero)
                      + jnp.where(s1 == eb, x1, zero))
        lane = jax.lax.broadcasted_iota(jnp.int32, (HT, DH), 1)
        q = jnp.where(lane == sel[:, 0:1], p0, 0.0) + jnp.where(
            lane == sel[:, 1:2], p1, 0.0)
        zs.append(q.astype(jnp.bfloat16))
        z = jnp.concatenate(zs, axis=1)
        o_ref[rows, :] = jnp.dot(z, ws_ref[...],
                                 preferred_element_type=jnp.float32)


@functools.partial(jax.jit, static_argnames=("interpret",))
def kernel(tensor, head_selection, head_probabilities, W, b, interpret=False):
    B, S, H, dh = tensor.shape
    N = B * S
    x = tensor.reshape(N, H * dh)
    sel = head_selection.reshape(N, H)
    p = head_probabilities.reshape(N, H)
    w2 = W.reshape(E * DH, DM)

    out = pl.pallas_call(
        _body,
        grid=(N // TILE,),
        in_specs=[
            pl.BlockSpec((TILE, H * dh), lambda i: (i, 0)),
            pl.BlockSpec((TILE, H), lambda i: (i, 0)),
            pl.BlockSpec((TILE, H), lambda i: (i, 0)),
            pl.BlockSpec((E * DH, DM), lambda i: (0, 0)),
            pl.BlockSpec((E, DM), lambda i: (0, 0)),
        ],
        out_specs=pl.BlockSpec((TILE, DM), lambda i: (i, 0)),
        out_shape=jax.ShapeDtypeStruct((N, DM), jnp.float32),
        scratch_shapes=[pltpu.VMEM((KAUG, DM), jnp.bfloat16)],
        compiler_params=pltpu.CompilerParams(
            vmem_limit_bytes=100 * 1024 * 1024),
        interpret=interpret,
    )(x, sel, p, w2, b)
    return out.reshape(B, S, DM)


# speedup vs baseline: 1.0441x; 1.0013x over previous
import jax
import jax.numpy as jnp
from jax.experimental import pallas as pl
from jax.experimental.pallas import tpu as pltpu

E = 8
DH = 128
DM = 2048
TILE = 1024
KAUG = E * DH + 128


def _body(x_ref, sel_ref, p_ref, w_ref, b_ref, o_ref, ws_ref):
    @pl.when(pl.program_id(0) == 0)
    def _init():
        def cast_chunk(k, _):
            ws_ref[pl.ds(k * DH, DH), :] = (
                w_ref[pl.ds(k * DH, DH), :].astype(jnp.bfloat16))
            return 0
        jax.lax.fori_loop(0, E, cast_chunk, 0)
        ws_ref[pl.ds(E * DH, DH), :] = jnp.concatenate(
            [b_ref[...], jnp.zeros((DH - E, DM), jnp.float32)], axis=0
        ).astype(jnp.bfloat16)

    HT = TILE // 2
    for h in range(2):
        rows = pl.ds(h * HT, HT)
        x = x_ref[rows, :]
        sel = sel_ref[rows, :]
        p = p_ref[rows, :]
        p0 = p[:, 0:1]
        p1 = p[:, 1:2]
        zero = jnp.zeros((), jnp.bfloat16)
        x0 = (x[:, :DH] * p0).astype(jnp.bfloat16)
        x1 = (x[:, DH:] * p1).astype(jnp.bfloat16)
        s0 = sel[:, 0:1].astype(jnp.bfloat16)
        s1 = sel[:, 1:2].astype(jnp.bfloat16)
        zs = []
        for e in range(E):
            eb = jnp.full((), e, jnp.bfloat16)
            zs.append(jnp.where(s0 == eb, x0, zero)
                      + jnp.where(s1 == eb, x1, zero))
        lane = jax.lax.broadcasted_iota(jnp.int32, (HT, DH), 1)
        q = jnp.where(lane == sel[:, 0:1], p0, 0.0) + jnp.where(
            lane == sel[:, 1:2], p1, 0.0)
        zs.append(q.astype(jnp.bfloat16))
        z = jnp.concatenate(zs, axis=1)
        o_ref[rows, :] = jnp.dot(z, ws_ref[...],
                                 preferred_element_type=jnp.float32)


@jax.jit
def kernel(tensor, head_selection, head_probabilities, W, b):
    B, S, H, dh = tensor.shape
    N = B * S
    x = tensor.reshape(N, H * dh)
    sel = head_selection.reshape(N, H)
    p = head_probabilities.reshape(N, H)
    w2 = W.reshape(E * DH, DM)

    out = pl.pallas_call(
        _body,
        grid=(N // TILE,),
        in_specs=[
            pl.BlockSpec((TILE, H * dh), lambda i: (i, 0)),
            pl.BlockSpec((TILE, H), lambda i: (i, 0)),
            pl.BlockSpec((TILE, H), lambda i: (i, 0)),
            pl.BlockSpec((E * DH, DM), lambda i: (0, 0)),
            pl.BlockSpec((E, DM), lambda i: (0, 0)),
        ],
        out_specs=pl.BlockSpec((TILE, DM), lambda i: (i, 0)),
        out_shape=jax.ShapeDtypeStruct((N, DM), jnp.float32),
        scratch_shapes=[pltpu.VMEM((KAUG, DM), jnp.bfloat16)],
        compiler_params=pltpu.CompilerParams(
            vmem_limit_bytes=100 * 1024 * 1024),
    )(x, sel, p, w2, b)
    return out.reshape(B, S, DM)
